# baseline (speedup 1.0000x reference)
import functools

import jax
import jax.numpy as jnp
from jax import lax
from jax.experimental import pallas as pl
from jax.experimental.pallas import tpu as pltpu
from jax.experimental.pallas import tpu_sc as plsc

N_ATOMS = 10000
N_EDGES = 320000
D = 128
N_RBF = 20

NC = 2
NS = 16
NW = NC * NS

CHUNK = 64
SUPCH = 40
SUP_EDGES = CHUNK * SUPCH
A_PAD = 10240
ROWS_PER_TILE = A_PAD // NS
IDX_ROWS = N_EDGES // CHUNK

_SPLITS = ((0, 102400), (102400, 107520), (209920, 110080))

_LN2 = 0.6931471805599453


def _ssp(x):
    return jnp.maximum(x, 0.0) + jnp.log1p(jnp.exp(-jnp.abs(x))) - _LN2


def _h_body(x_ref, w_ref, o_ref):
    o_ref[...] = jnp.dot(x_ref[...], w_ref[...],
                         preferred_element_type=jnp.float32)


def _compute_h(x, W_in2f):
    return pl.pallas_call(
        _h_body,
        out_shape=jax.ShapeDtypeStruct((N_ATOMS, D), jnp.float32),
    )(x, W_in2f)


_EBLK = 5120
_RROWS = _EBLK // D


def _wij_body(f_ref, r_ref, sel_ref, mask_ref, ones_ref,
              w1_ref, b1_ref, w2_ref, b2_ref, o_ref):
    t = lax.dot_general(f_ref[...], w1_ref[...], (((0,), (0,)), ((), ())),
                        preferred_element_type=jnp.float32)
    t = _ssp(t + b1_ref[...])
    w = jnp.dot(t, w2_ref[...], preferred_element_type=jnp.float32) + b2_ref[...]
    o = jnp.dot(sel_ref[...], r_ref[...], preferred_element_type=jnp.float32,
                precision=lax.Precision.HIGHEST)
    rbc = jnp.dot(o * mask_ref[...], ones_ref[...],
                  preferred_element_type=jnp.float32,
                  precision=lax.Precision.HIGHEST)
    o_ref[...] = w * rbc


def _compute_wij(f_ij, rcut, W_f1, b_f1, W_f2, b_f2, e_start, n_e):
    off = e_start // _EBLK
    return pl.pallas_call(
        _wij_body,
        grid=(pl.cdiv(n_e, _EBLK),),
        in_specs=[
            pl.BlockSpec((N_RBF, _EBLK), lambda i: (0, off + i)),
            pl.BlockSpec((_RROWS, D), lambda i: (off + i, 0)),
            pl.BlockSpec((_EBLK, _RROWS), lambda i: (0, 0)),
            pl.BlockSpec((_EBLK, D), lambda i: (0, 0)),
            pl.BlockSpec((D, 1), lambda i: (0, 0)),
            pl.BlockSpec((N_RBF, D), lambda i: (0, 0)),
            pl.BlockSpec((1, D), lambda i: (0, 0)),
            pl.BlockSpec((D, D), lambda i: (0, 0)),
            pl.BlockSpec((1, D), lambda i: (0, 0)),
        ],
        out_specs=pl.BlockSpec((_EBLK, D), lambda i: (i, 0)),
        out_shape=jax.ShapeDtypeStruct((n_e, D), jnp.float32),
    )(f_ij.T, rcut.reshape(N_EDGES // D, D), _sel_const(), _mask_const(),
      jnp.ones((D, 1), jnp.float32), W_f1, b_f1.reshape(1, D), W_f2,
      b_f2.reshape(1, D))


def _sel_const():
    e = jnp.arange(_EBLK, dtype=jnp.int32)
    k = jnp.arange(_RROWS, dtype=jnp.int32)
    return jnp.where(e[:, None] // D == k[None, :], 1.0, 0.0)


def _mask_const():
    e = jnp.arange(_EBLK, dtype=jnp.int32)
    l = jnp.arange(D, dtype=jnp.int32)
    return jnp.where(l[None, :] == e[:, None] % D, 1.0, 0.0)


def _sc_conv_body(e_start, n_sup_total,
                  h_hbm, wij_hbm, idxi_hbm, idxj_hbm, zeros_hbm,
                  out_hbm, idxj_v, idxi_v, xj_v, w_v, conv_sh,
                  gsem0, gsem1, wsem0, wsem1, ssem0, ssem1, isem):
    c = lax.axis_index("c")
    s = lax.axis_index("s")
    wid = c * NS + s
    gsems = (gsem0, gsem1)
    wsems = (wsem0, wsem1)
    ssems = (ssem0, ssem1)

    r0 = s * ROWS_PER_TILE
    pltpu.sync_copy(zeros_hbm.at[pl.ds(r0, ROWS_PER_TILE)],
                    conv_sh.at[pl.ds(r0, ROWS_PER_TILE)])
    plsc.subcore_barrier()

    n_sup_w = jnp.maximum(0, (n_sup_total - wid + NW - 1) // NW)
    start_row = e_start // CHUNK

    def start_chunk(base_e, kk, b):
        e0 = pl.multiple_of(base_e + kk * CHUNK, CHUNK)
        pltpu.async_copy(h_hbm.at[idxj_v.at[kk]], xj_v.at[b], gsems[b])
        pltpu.async_copy(wij_hbm.at[pl.ds(e0, CHUNK)], w_v.at[b], wsems[b])

    def superstep(t, carry):
        ss = wid + t * NW
        base_e = ss * SUP_EDGES
        row0 = pl.multiple_of(start_row + ss * SUPCH, 8)
        cp = pltpu.async_copy(idxj_hbm.at[pl.ds(row0, SUPCH)], idxj_v, isem)
        pltpu.sync_copy(idxi_hbm.at[pl.ds(row0, SUPCH)], idxi_v)
        cp.wait()

        @pl.when(t > 0)
        def _():
            pltpu.make_async_copy(zeros_hbm.at[pl.ds(0, CHUNK)],
                                  xj_v.at[0], ssems[0]).wait()
            pltpu.make_async_copy(zeros_hbm.at[pl.ds(0, CHUNK)],
                                  xj_v.at[1], ssems[1]).wait()

        start_chunk(base_e, 0, 0)
        for kk in range(SUPCH):
            b = kk % 2
            b2 = 1 - b
            pltpu.make_async_copy(zeros_hbm.at[pl.ds(0, CHUNK)],
                                  xj_v.at[b], gsems[b]).wait()
            pltpu.make_async_copy(zeros_hbm.at[pl.ds(0, CHUNK)],
                                  w_v.at[b], wsems[b]).wait()

            if kk + 1 < SUPCH:
                if kk >= 1:
                    pltpu.make_async_copy(zeros_hbm.at[pl.ds(0, CHUNK)],
                                          xj_v.at[b2], ssems[b2]).wait()
                start_chunk(base_e, kk + 1, b2)

            @plsc.parallel_loop(0, CHUNK, unroll=4)
            def _(r):
                for cc in range(D // 16):
                    sl = pl.ds(cc * 16, 16)
                    xj_v[b, r, sl] = xj_v[b, r, sl] * w_v[b, r, sl]

            pltpu.async_copy(xj_v.at[b], conv_sh.at[idxi_v.at[kk]],
                             ssems[b], add=True)
        return carry

    lax.fori_loop(0, n_sup_w, superstep, 0)

    @pl.when(n_sup_w > 0)
    def _():
        pltpu.make_async_copy(zeros_hbm.at[pl.ds(0, CHUNK)],
                              xj_v.at[0], ssems[0]).wait()
        pltpu.make_async_copy(zeros_hbm.at[pl.ds(0, CHUNK)],
                              xj_v.at[1], ssems[1]).wait()

    plsc.subcore_barrier()
    pltpu.sync_copy(conv_sh.at[pl.ds(r0, ROWS_PER_TILE)],
                    out_hbm.at[c, pl.ds(r0, ROWS_PER_TILE)])


@functools.cache
def _make_sc_conv(e_start, n_sup_total):
    mesh = plsc.VectorSubcoreMesh(core_axis_name="c", subcore_axis_name="s")
    return pl.kernel(
        functools.partial(_sc_conv_body, e_start, n_sup_total),
        out_type=jax.ShapeDtypeStruct((NC, A_PAD, D), jnp.float32),
        mesh=mesh,
        scratch_types=[
            pltpu.VMEM((SUPCH, CHUNK), jnp.int32),
            pltpu.VMEM((SUPCH, CHUNK), jnp.int32),
            pltpu.VMEM((2, CHUNK, D), jnp.float32),
            pltpu.VMEM((2, CHUNK, D), jnp.float32),
            pltpu.VMEM_SHARED((A_PAD, D), jnp.float32),
            pltpu.SemaphoreType.DMA,
            pltpu.SemaphoreType.DMA,
            pltpu.SemaphoreType.DMA,
            pltpu.SemaphoreType.DMA,
            pltpu.SemaphoreType.DMA,
            pltpu.SemaphoreType.DMA,
            pltpu.SemaphoreType.DMA,
        ],
    )


_OBLK = 1000


def _out_body(p0_ref, p1_ref, p2_ref, w1_ref, b1_ref, w2_ref, b2_ref, o_ref):
    acc = (p0_ref[0] + p0_ref[1] + p1_ref[0] + p1_ref[1]
           + p2_ref[0] + p2_ref[1])
    t = _ssp(jnp.dot(acc, w1_ref[...], preferred_element_type=jnp.float32)
             + b1_ref[...])
    o_ref[...] = jnp.dot(t, w2_ref[...],
                         preferred_element_type=jnp.float32) + b2_ref[...]


def _compute_out(partials, W_o1, b_o1, W_o2, b_o2):
    grid = N_ATOMS // _OBLK
    pspec = pl.BlockSpec((NC, _OBLK, D), lambda i: (0, i, 0))
    return pl.pallas_call(
        _out_body,
        grid=(grid,),
        in_specs=[
            pspec, pspec, pspec,
            pl.BlockSpec((D, D), lambda i: (0, 0)),
            pl.BlockSpec((1, D), lambda i: (0, 0)),
            pl.BlockSpec((D, D), lambda i: (0, 0)),
            pl.BlockSpec((1, D), lambda i: (0, 0)),
        ],
        out_specs=pl.BlockSpec((_OBLK, D), lambda i: (i, 0)),
        out_shape=jax.ShapeDtypeStruct((N_ATOMS, D), jnp.float32),
    )(*partials, W_o1, b_o1.reshape(1, D), W_o2, b_o2.reshape(1, D))


def kernel(x, f_ij, idx_i, idx_j, rcut_ij,
           W_in2f, W_f1, b_f1, W_f2, b_f2, W_o1, b_o1, W_o2, b_o2):
    idxi_2d = idx_i.reshape(IDX_ROWS, CHUNK)
    idxj_2d = idx_j.reshape(IDX_ROWS, CHUNK)
    zeros = jnp.zeros((A_PAD, D), jnp.float32)

    h = _compute_h(x, W_in2f)
    partials = []
    for e_start, n_e in _SPLITS:
        wij = _compute_wij(f_ij, rcut_ij, W_f1, b_f1, W_f2, b_f2, e_start, n_e)
        partials.append(
            _make_sc_conv(e_start, n_e // SUP_EDGES)(
                h, wij, idxi_2d, idxj_2d, zeros))
    return _compute_out(partials, W_o1, b_o1, W_o2, b_o2)


# speedup vs baseline: 1.7280x; 1.7280x over previous
import functools

import jax
import jax.numpy as jnp
from jax import lax
from jax.experimental import pallas as pl
from jax.experimental.pallas import tpu as pltpu
from jax.experimental.pallas import tpu_sc as plsc

N_ATOMS = 10000
N_EDGES = 320000
D = 128
N_RBF = 20

NC = 2
NS = 16
NW = NC * NS

CHUNK = 64
SUPCH = 40
SUP_EDGES = CHUNK * SUPCH
A_PAD = 10240
ROWS_PER_TILE = A_PAD // NS
IDX_ROWS = N_EDGES // CHUNK

_E_SPLIT = 158720
_HALVES = ((0, _E_SPLIT), (_E_SPLIT, N_EDGES - _E_SPLIT))

_LN2 = 0.6931471805599453


def _ssp(x):
    return jnp.maximum(x, 0.0) + jnp.log1p(jnp.exp(-jnp.abs(x))) - _LN2


def _h_body(x_ref, w_ref, o_ref):
    o_ref[...] = jnp.dot(x_ref[...], w_ref[...],
                         preferred_element_type=jnp.float32)


def _compute_h(x, W_in2f):
    return pl.pallas_call(
        _h_body,
        out_shape=jax.ShapeDtypeStruct((N_ATOMS, D), jnp.float32),
    )(x, W_in2f)


_EBLK = 5120
_RROWS = _EBLK // D


def _wij_body(f_ref, r_ref, w1_ref, b1_ref, w2_ref, b2_ref, o_ref):
    t = lax.dot_general(f_ref[...], w1_ref[...], (((0,), (0,)), ((), ())),
                        preferred_element_type=jnp.float32)
    t = _ssp(t + b1_ref[...])
    w = jnp.dot(t, w2_ref[...], preferred_element_type=jnp.float32) + b2_ref[...]
    sub = lax.broadcasted_iota(jnp.int32, (_EBLK, _RROWS), 0)
    kk = lax.broadcasted_iota(jnp.int32, (_EBLK, _RROWS), 1)
    sel = jnp.where(sub // D == kk, 1.0, 0.0)
    o = jnp.dot(sel, r_ref[...], preferred_element_type=jnp.float32,
                precision=lax.Precision.HIGHEST)
    lane = lax.broadcasted_iota(jnp.int32, (_EBLK, D), 1)
    subm = lax.broadcasted_iota(jnp.int32, (_EBLK, D), 0) % D
    rbc = jnp.sum(jnp.where(lane == subm, o, 0.0), axis=1, keepdims=True)
    o_ref[...] = w * rbc


def _compute_wij(f_ij, rcut, W_f1, b_f1, W_f2, b_f2, e_start, n_e):
    off = e_start // _EBLK
    return pl.pallas_call(
        _wij_body,
        grid=(pl.cdiv(n_e, _EBLK),),
        in_specs=[
            pl.BlockSpec((N_RBF, _EBLK), lambda i: (0, off + i)),
            pl.BlockSpec((_RROWS, D), lambda i: (off + i, 0)),
            pl.BlockSpec((N_RBF, D), lambda i: (0, 0)),
            pl.BlockSpec((1, D), lambda i: (0, 0)),
            pl.BlockSpec((D, D), lambda i: (0, 0)),
            pl.BlockSpec((1, D), lambda i: (0, 0)),
        ],
        out_specs=pl.BlockSpec((_EBLK, D), lambda i: (i, 0)),
        out_shape=jax.ShapeDtypeStruct((n_e, D), jnp.float32),
    )(f_ij.T, rcut.reshape(N_EDGES // D, D), W_f1, b_f1.reshape(1, D), W_f2,
      b_f2.reshape(1, D))


def _sc_conv_body(e_start, n_sup_total,
                  h_hbm, wij_hbm, idxi_hbm, idxj_hbm, zeros_hbm,
                  out_hbm, idxj_v, idxi_v, xj_v, w_v, conv_sh,
                  gsem0, gsem1, wsem0, wsem1, ssem0, ssem1, isem):
    c = lax.axis_index("c")
    s = lax.axis_index("s")
    wid = c * NS + s
    gsems = (gsem0, gsem1)
    wsems = (wsem0, wsem1)
    ssems = (ssem0, ssem1)

    r0 = s * ROWS_PER_TILE
    pltpu.sync_copy(zeros_hbm.at[pl.ds(r0, ROWS_PER_TILE)],
                    conv_sh.at[pl.ds(r0, ROWS_PER_TILE)])
    plsc.subcore_barrier()

    n_sup_w = jnp.maximum(0, (n_sup_total - wid + NW - 1) // NW)
    start_row = e_start // CHUNK

    def start_chunk(base_e, kk, b):
        e0 = pl.multiple_of(base_e + kk * CHUNK, CHUNK)
        pltpu.async_copy(h_hbm.at[idxj_v.at[kk]], xj_v.at[b], gsems[b])
        pltpu.async_copy(wij_hbm.at[pl.ds(e0, CHUNK)], w_v.at[b], wsems[b])

    def superstep(t, carry):
        ss = wid + t * NW
        base_e = ss * SUP_EDGES
        row0 = pl.multiple_of(start_row + ss * SUPCH, 8)
        cp = pltpu.async_copy(idxj_hbm.at[pl.ds(row0, SUPCH)], idxj_v, isem)
        pltpu.sync_copy(idxi_hbm.at[pl.ds(row0, SUPCH)], idxi_v)
        cp.wait()

        @pl.when(t > 0)
        def _():
            pltpu.make_async_copy(zeros_hbm.at[pl.ds(0, CHUNK)],
                                  xj_v.at[0], ssems[0]).wait()
            pltpu.make_async_copy(zeros_hbm.at[pl.ds(0, CHUNK)],
                                  xj_v.at[1], ssems[1]).wait()

        start_chunk(base_e, 0, 0)
        for kk in range(SUPCH):
            b = kk % 2
            b2 = 1 - b
            pltpu.make_async_copy(zeros_hbm.at[pl.ds(0, CHUNK)],
                                  xj_v.at[b], gsems[b]).wait()
            pltpu.make_async_copy(zeros_hbm.at[pl.ds(0, CHUNK)],
                                  w_v.at[b], wsems[b]).wait()

            if kk + 1 < SUPCH:
                if kk >= 1:
                    pltpu.make_async_copy(zeros_hbm.at[pl.ds(0, CHUNK)],
                                          xj_v.at[b2], ssems[b2]).wait()
                start_chunk(base_e, kk + 1, b2)

            @plsc.parallel_loop(0, CHUNK, unroll=4)
            def _(r):
                for cc in range(D // 16):
                    sl = pl.ds(cc * 16, 16)
                    xj_v[b, r, sl] = xj_v[b, r, sl] * w_v[b, r, sl]

            pltpu.async_copy(xj_v.at[b], conv_sh.at[idxi_v.at[kk]],
                             ssems[b], add=True)
        return carry

    lax.fori_loop(0, n_sup_w, superstep, 0)

    @pl.when(n_sup_w > 0)
    def _():
        pltpu.make_async_copy(zeros_hbm.at[pl.ds(0, CHUNK)],
                              xj_v.at[0], ssems[0]).wait()
        pltpu.make_async_copy(zeros_hbm.at[pl.ds(0, CHUNK)],
                              xj_v.at[1], ssems[1]).wait()

    plsc.subcore_barrier()
    pltpu.sync_copy(conv_sh.at[pl.ds(r0, ROWS_PER_TILE)],
                    out_hbm.at[c, pl.ds(r0, ROWS_PER_TILE)])


@functools.cache
def _make_sc_conv(e_start, n_sup_total):
    mesh = plsc.VectorSubcoreMesh(core_axis_name="c", subcore_axis_name="s")
    return pl.kernel(
        functools.partial(_sc_conv_body, e_start, n_sup_total),
        out_type=jax.ShapeDtypeStruct((NC, A_PAD, D), jnp.float32),
        mesh=mesh,
        scratch_types=[
            pltpu.VMEM((SUPCH, CHUNK), jnp.int32),
            pltpu.VMEM((SUPCH, CHUNK), jnp.int32),
            pltpu.VMEM((2, CHUNK, D), jnp.float32),
            pltpu.VMEM((2, CHUNK, D), jnp.float32),
            pltpu.VMEM_SHARED((A_PAD, D), jnp.float32),
            pltpu.SemaphoreType.DMA,
            pltpu.SemaphoreType.DMA,
            pltpu.SemaphoreType.DMA,
            pltpu.SemaphoreType.DMA,
            pltpu.SemaphoreType.DMA,
            pltpu.SemaphoreType.DMA,
            pltpu.SemaphoreType.DMA,
        ],
    )


_OBLK = 1000


def _out_body(p0_ref, p1_ref, w1_ref, b1_ref, w2_ref, b2_ref, o_ref):
    acc = p0_ref[0] + p0_ref[1] + p1_ref[0] + p1_ref[1]
    t = _ssp(jnp.dot(acc, w1_ref[...], preferred_element_type=jnp.float32)
             + b1_ref[...])
    o_ref[...] = jnp.dot(t, w2_ref[...],
                         preferred_element_type=jnp.float32) + b2_ref[...]


def _compute_out(p0, p1, W_o1, b_o1, W_o2, b_o2):
    grid = N_ATOMS // _OBLK
    return pl.pallas_call(
        _out_body,
        grid=(grid,),
        in_specs=[
            pl.BlockSpec((NC, _OBLK, D), lambda i: (0, i, 0)),
            pl.BlockSpec((NC, _OBLK, D), lambda i: (0, i, 0)),
            pl.BlockSpec((D, D), lambda i: (0, 0)),
            pl.BlockSpec((1, D), lambda i: (0, 0)),
            pl.BlockSpec((D, D), lambda i: (0, 0)),
            pl.BlockSpec((1, D), lambda i: (0, 0)),
        ],
        out_specs=pl.BlockSpec((_OBLK, D), lambda i: (i, 0)),
        out_shape=jax.ShapeDtypeStruct((N_ATOMS, D), jnp.float32),
    )(p0, p1, W_o1, b_o1.reshape(1, D), W_o2, b_o2.reshape(1, D))


def kernel(x, f_ij, idx_i, idx_j, rcut_ij,
           W_in2f, W_f1, b_f1, W_f2, b_f2, W_o1, b_o1, W_o2, b_o2):
    idxi_2d = idx_i.reshape(IDX_ROWS, CHUNK)
    idxj_2d = idx_j.reshape(IDX_ROWS, CHUNK)
    zeros = jnp.zeros((A_PAD, D), jnp.float32)

    h = _compute_h(x, W_in2f)
    partials = []
    for e_start, n_e in _HALVES:
        wij = _compute_wij(f_ij, rcut_ij, W_f1, b_f1, W_f2, b_f2, e_start, n_e)
        partials.append(
            _make_sc_conv(e_start, n_e // SUP_EDGES)(
                h, wij, idxi_2d, idxj_2d, zeros))
    return _compute_out(partials[0], partials[1], W_o1, b_o1, W_o2, b_o2)
